# Initial kernel scaffold; baseline (speedup 1.0000x reference)
import jax
import jax.numpy as jnp
from jax.experimental import pallas as pl


def kernel(vertices, edges, faces, edge_to_vertex, face_to_edge, face_to_face, Wv, bv, We, be, Wf, bf, Wv2e, bv2e, We2f, be2f, Wm0, bm0, Wm1, bm1, Wm2, bm2):
    raise NotImplementedError("write your pallas kernel here")


# speedup vs baseline: 3.5330x; 3.5330x over previous
import functools

import jax
import jax.numpy as jnp
from jax import lax
from jax.experimental import pallas as pl
from jax.experimental.pallas import tpu as pltpu
from jax.experimental.pallas import tpu_sc as plsc

F32 = jnp.float32
I32 = jnp.int32

N = 10000
D = 256
NC, NS, L = 2, 16, 16
NW = NC * NS
RPW = 320
NPAD = NW * RPW
ACC = RPW * D
CAP = 12288
CAPP = CAP + 64
EG = 16
BIG = 3.0e38
CH = 4000

_MESH = plsc.VectorSubcoreMesh(core_axis_name="c", subcore_axis_name="s")
_SC_PARAMS = pltpu.CompilerParams(needs_layout_passes=False)


def _wid():
    return lax.axis_index("s") * NC + lax.axis_index("c")


def _make_filter(E):
    n_chunks = -(-E // CH)
    assert E % 16 == 0

    def body(dst_hbm, src_hbm, srcl_out, dstl_out, cnt_out,
             dstc, srcc, srclv, dstlv, cntv):
        w = _wid()
        lo = w * RPW
        hi = lo + RPW

        def chunk(g, count):
            sz = CH if E % CH == 0 else CH
            pltpu.sync_copy(dst_hbm.at[pl.ds(g * CH, sz)], dstc)
            pltpu.sync_copy(src_hbm.at[pl.ds(g * CH, sz)], srcc)

            def grp(i, count):
                d16 = dstc[pl.ds(i * L, L)]
                s16 = srcc[pl.ds(i * L, L)]
                m = (d16 >= lo) & (d16 < hi)
                c = plsc.cumsum(m.astype(I32))
                pos = count - 1 + c
                plsc.store_scatter(srclv, [pos], s16, mask=m)
                plsc.store_scatter(dstlv, [pos], d16 - lo, mask=m)
                return count + jnp.max(c)

            return lax.fori_loop(0, CH // L, grp, count)

        count = lax.fori_loop(0, n_chunks, chunk, jnp.int32(0))

        sent_d = jnp.full((L,), RPW, I32)
        sent_s = jnp.zeros((L,), I32)
        for k in range(4):
            dstlv[pl.ds(count + k * L, L)] = sent_d
            srclv[pl.ds(count + k * L, L)] = sent_s
        cnt_p = jnp.maximum(jnp.int32(64), ((count + 63) // 64) * 64)
        cntv[...] = jnp.full((L,), cnt_p, I32)
        pltpu.sync_copy(srclv, srcl_out.at[w])
        pltpu.sync_copy(dstlv, dstl_out.at[w])
        pltpu.sync_copy(cntv, cnt_out.at[w])

    return pl.kernel(
        body,
        out_type=(jax.ShapeDtypeStruct((NW, CAPP), I32),
                  jax.ShapeDtypeStruct((NW, CAPP), I32),
                  jax.ShapeDtypeStruct((NW, L), I32)),
        mesh=_MESH,
        compiler_params=_SC_PARAMS,
        scratch_types=[pltpu.VMEM((CH,), I32),
                       pltpu.VMEM((CH,), I32),
                       pltpu.VMEM((CAPP,), I32),
                       pltpu.VMEM((CAPP,), I32),
                       pltpu.VMEM((L,), I32)],
    )


def _segmin_body(table, srcl, dstl, cnt, seg_out,
                 acc, srclv, dstlv, cntv, rows, sem0, sem1):
    w = _wid()
    pltpu.sync_copy(cnt.at[w], cntv)
    pltpu.sync_copy(srcl.at[w], srclv)
    pltpu.sync_copy(dstl.at[w], dstlv)
    n = jnp.max(cntv[...])

    big = jnp.full((L,), BIG, F32)

    def ini(i, _):
        for k in range(D // L):
            acc[pl.ds(i * D + k * L, L)] = big
        return 0

    lax.fori_loop(0, RPW + 1, ini, 0)

    sems = (sem0, sem1)
    for b in range(2):
        pltpu.async_copy(table.at[srclv.at[pl.ds(b * EG, EG)]],
                         rows.at[b], sems[b])

    npair = n // (2 * EG)

    def pair(p, _):
        for b in range(2):
            g = 2 * p + b
            pltpu.make_async_copy(table.at[srclv.at[pl.ds(0, EG)]],
                                  rows.at[b], sems[b]).wait()
            dvec = dstlv[pl.ds(g * EG, L)] * D
            for j in range(L):
                off = dvec[j]
                for k in range(D // L):
                    sl = pl.ds(off + k * L, L)
                    acc[sl] = jnp.minimum(acc[sl], rows[b, j, pl.ds(k * L, L)])

            @pl.when(g + 2 < 2 * npair)
            def _(g=g, b=b):
                pltpu.async_copy(table.at[srclv.at[pl.ds((g + 2) * EG, EG)]],
                                 rows.at[b], sems[b])
        return 0

    lax.fori_loop(0, npair, pair, 0)
    pltpu.sync_copy(acc.at[pl.ds(0, ACC)], seg_out.at[w])


_SEGMIN = pl.kernel(
    _segmin_body,
    out_type=jax.ShapeDtypeStruct((NW, ACC), F32),
    mesh=_MESH,
    compiler_params=_SC_PARAMS,
    scratch_types=[pltpu.VMEM((ACC + D,), F32),
                   pltpu.VMEM((CAPP,), I32),
                   pltpu.VMEM((CAPP,), I32),
                   pltpu.VMEM((L,), I32),
                   pltpu.VMEM((2, EG, D), F32),
                   pltpu.SemaphoreType.DMA,
                   pltpu.SemaphoreType.DMA],
)


BLKE = 1024
BLKM = 512


def _enc_body(v_ref, e_ref, f_ref, wv_ref, we_ref, wf_ref,
              bv_ref, be_ref, bf_ref, xv_ref, xe_ref, xf_ref):
    for x_ref, w_ref, b_ref, o_ref in ((v_ref, wv_ref, bv_ref, xv_ref),
                                       (e_ref, we_ref, be_ref, xe_ref),
                                       (f_ref, wf_ref, bf_ref, xf_ref)):
        y = jnp.dot(x_ref[...], w_ref[...], preferred_element_type=F32)
        y = y + b_ref[...]
        y = jnp.where(y >= 0, y, 0.01 * y)
        y = jnp.where(jnp.isnan(y), 0.0, y)
        o_ref[...] = jnp.clip(y, -10000.0, 10000.0)


def _encoders(v8, e8, f8, wv, we, wf, bv, be, bf):
    row = pl.BlockSpec((BLKE, 8), lambda i: (i, 0))
    full = pl.BlockSpec((8, D), lambda i: (0, 0))
    bias = pl.BlockSpec((1, D), lambda i: (0, 0))
    out = pl.BlockSpec((BLKE, D), lambda i: (i, 0))
    return pl.pallas_call(
        _enc_body,
        grid=(NPAD // BLKE,),
        in_specs=[row, row, row, full, full, full, bias, bias, bias],
        out_specs=[out, out, out],
        out_shape=[jax.ShapeDtypeStruct((NPAD, D), F32)] * 3,
    )(v8, e8, f8, wv, we, wf, bv, be, bf)


def _make_mlp(self_loop):
    def body(x_ref, s_ref, w0_ref, w1_ref, b_ref, o_ref):
        x = x_ref[...]
        s = s_ref[...]
        if self_loop:
            mx = x - jnp.minimum(s, x)
        else:
            mx = jnp.where(s > 1e30, 0.0, x - s)
        y = jnp.dot(x, w0_ref[...], preferred_element_type=F32)
        y = y + jnp.dot(mx, w1_ref[...], preferred_element_type=F32)
        y = y + b_ref[...]
        y = jnp.where(y >= 0, y, 0.01 * y)
        y = x + y
        y = jnp.where(jnp.isnan(y), 0.0, y)
        o_ref[...] = jnp.clip(y, -10000.0, 10000.0)

    row = pl.BlockSpec((BLKM, D), lambda i: (i, 0))
    wsp = pl.BlockSpec((D, D), lambda i: (0, 0))
    bias = pl.BlockSpec((1, D), lambda i: (0, 0))

    def run(x, s, w, b):
        return pl.pallas_call(
            body,
            grid=(NPAD // BLKM,),
            in_specs=[row, row, wsp, wsp, bias],
            out_specs=row,
            out_shape=jax.ShapeDtypeStruct((NPAD, D), F32),
        )(x, s, w[:D], w[D:], b.reshape(1, D))

    return run


_MLP_PLAIN = _make_mlp(False)
_MLP_LOOP = _make_mlp(True)


def _pad_feat(x):
    out = jnp.zeros((NPAD, 8), F32)
    return out.at[:x.shape[0], :x.shape[1]].set(x)


def _pad_w(w):
    out = jnp.zeros((8, D), F32)
    return out.at[:w.shape[0]].set(w)


def kernel(vertices, edges, faces, edge_to_vertex, face_to_edge, face_to_face,
           Wv, bv, We, be, Wf, bf, Wv2e, bv2e, We2f, be2f,
           Wm0, bm0, Wm1, bm1, Wm2, bm2):
    x_v, x_e, x_f = _encoders(
        _pad_feat(vertices), _pad_feat(edges), _pad_feat(faces),
        _pad_w(Wv), _pad_w(We), _pad_w(Wf),
        bv.reshape(1, D), be.reshape(1, D), bf.reshape(1, D))

    ev = _make_filter(edge_to_vertex.shape[1])(
        edge_to_vertex[0], edge_to_vertex[1])
    fe = _make_filter(face_to_edge.shape[1])(
        face_to_edge[0], face_to_edge[1])
    ff = _make_filter(face_to_face.shape[1])(
        face_to_face[1], face_to_face[0])

    def seg(table, lists):
        s = _SEGMIN(table, *lists)
        return s.reshape(NPAD, D)

    x_e = _MLP_PLAIN(x_e, seg(x_v, ev), Wv2e, bv2e)
    x_f = _MLP_PLAIN(x_f, seg(x_e, fe), We2f, be2f)
    for w, b in ((Wm0, bm0), (Wm1, bm1), (Wm2, bm2)):
        x_f = _MLP_LOOP(x_f, seg(x_f, ff), w, b)
    return x_f[:N]
